# baseline (speedup 1.0000x reference)
import functools

import jax
import jax.numpy as jnp
from jax import lax
from jax.experimental import pallas as pl
from jax.experimental.pallas import tpu as pltpu
from jax.experimental.pallas import tpu_sc as plsc

N_POS = 2048
N_SPECIES = 60
N_AA = 23
N_VOXEL = 4096
D = 6 * N_SPECIES
D_VAL = 384
D_TBL = 768
D_OUT = N_AA * D
D_OUT_PAD = 8320

_NC = 2
_NS = 16
_NW = _NC * _NS
_BPW = N_VOXEL // _NW

_TC_B = 128


def _sc_gather_body(tbl_hbm, idx_hbm, out_hbm, idx_v, rows_v, sem):
    wid = lax.axis_index("s") * _NC + lax.axis_index("c")
    base = wid * _BPW
    pltpu.sync_copy(idx_hbm.at[pl.ds(base, _BPW)], idx_v)
    pltpu.async_copy(tbl_hbm.at[idx_v], rows_v, sem).wait()
    pltpu.sync_copy(rows_v, out_hbm.at[pl.ds(base, _BPW)])


def _sc_gather(tbl, idx):
    mesh = plsc.VectorSubcoreMesh(core_axis_name="c", subcore_axis_name="s")
    k = functools.partial(
        pl.kernel,
        mesh=mesh,
        out_type=jax.ShapeDtypeStruct((N_VOXEL, D_TBL), jnp.float32),
        scratch_types=[
            pltpu.VMEM((_BPW,), jnp.int32),
            pltpu.VMEM((_BPW, D_TBL), jnp.float32),
            pltpu.SemaphoreType.DMA,
        ],
    )(_sc_gather_body)
    return k(tbl, idx)


def _tc_expand_body(g_ref, vl_ref, out_ref):
    seq = g_ref[:, :D]
    val = g_ref[:, D_VAL:D_VAL + D]
    vl = vl_ref[...]
    col = lax.broadcasted_iota(jnp.int32, (seq.shape[0], D), 1)
    scaled = ((col >= N_SPECIES) & (col < 2 * N_SPECIES)) | (col >= 5 * N_SPECIES)
    val = val * jnp.where(scaled, jnp.float32(1.0 / 254.0), jnp.float32(1.0))
    val = jnp.where(vl != -1, val, jnp.float32(0.0))
    seqi = seq.astype(jnp.int32)
    for a in range(N_AA):
        out_ref[:, a * D:(a + 1) * D] = jnp.where(
            seqi == a, val, jnp.float32(0.0))
    out_ref[:, D_OUT:] = jnp.zeros((seq.shape[0], D_OUT_PAD - D_OUT), jnp.float32)


_N_CHUNK = 4
_CHUNK = N_VOXEL // _N_CHUNK


def _tc_expand_chunk(g, vl2d, c):
    nb = _CHUNK // _TC_B
    return pl.pallas_call(
        _tc_expand_body,
        grid=(nb,),
        in_specs=[
            pl.BlockSpec((_TC_B, D_TBL), lambda i: (c * nb + i, 0)),
            pl.BlockSpec((_TC_B, 1), lambda i: (c * nb + i, 0)),
        ],
        out_specs=pl.BlockSpec((_TC_B, D_OUT_PAD), lambda i: (i, 0)),
        out_shape=jax.ShapeDtypeStruct((_CHUNK, D_OUT_PAD), jnp.float32),
    )(g, vl2d)


def kernel(seqArr, gapArr, stopArr, globalArr, voxel_local):
    seqf = seqArr.astype(jnp.float32)
    g = globalArr[0]
    ones = jnp.ones((N_POS, N_SPECIES), jnp.float32)
    z24 = jnp.zeros((N_POS, D_VAL - D), jnp.float32)
    tbl = jnp.concatenate(
        [
            seqf, seqf, seqf, seqf, seqf, seqf,
            z24,
            ones,
            gapArr,
            stopArr,
            jnp.broadcast_to(g[0][None, :], (N_POS, N_SPECIES)),
            jnp.broadcast_to(g[1][None, :], (N_POS, N_SPECIES)),
            jnp.broadcast_to(g[2][None, :], (N_POS, N_SPECIES)),
            z24,
        ],
        axis=1,
    )
    safe_idx = jnp.clip(voxel_local, 0, N_POS - 1)
    gathered = _sc_gather(tbl, safe_idx)
    vl2d = voxel_local.reshape(N_VOXEL, 1)
    out = jnp.empty((N_VOXEL, D_OUT), jnp.float32)
    for c in range(_N_CHUNK):
        part = _tc_expand_chunk(gathered, vl2d, c)[:, :D_OUT]
        out = lax.dynamic_update_slice(out, part, (c * _CHUNK, 0))
    return out


# speedup vs baseline: 1.6683x; 1.6683x over previous
import functools

import jax
import jax.numpy as jnp
from jax import lax
from jax.experimental import pallas as pl
from jax.experimental.pallas import tpu as pltpu
from jax.experimental.pallas import tpu_sc as plsc

N_POS = 2048
N_SPECIES = 60
N_AA = 23
N_VOXEL = 4096
D = 6 * N_SPECIES
D_VAL = 384
D_TBL = 768
D_OUT = N_AA * D
D_OUT_PAD = 8320

_NC = 2
_NS = 16
_NW = _NC * _NS
_BPW = N_VOXEL // _NW

_TC_B = 128


def _sc_gather_body(tbl_hbm, idx_hbm, out_hbm, idx_v, rows_v, sem):
    wid = lax.axis_index("s") * _NC + lax.axis_index("c")
    base = wid * _BPW
    pltpu.sync_copy(idx_hbm.at[pl.ds(base, _BPW)], idx_v)
    pltpu.async_copy(tbl_hbm.at[idx_v], rows_v, sem).wait()
    pltpu.sync_copy(rows_v, out_hbm.at[pl.ds(base, _BPW)])


def _sc_gather(tbl, idx):
    mesh = plsc.VectorSubcoreMesh(core_axis_name="c", subcore_axis_name="s")
    k = functools.partial(
        pl.kernel,
        mesh=mesh,
        out_type=jax.ShapeDtypeStruct((N_VOXEL, D_TBL), jnp.float32),
        scratch_types=[
            pltpu.VMEM((_BPW,), jnp.int32),
            pltpu.VMEM((_BPW, D_TBL), jnp.float32),
            pltpu.SemaphoreType.DMA,
        ],
    )(_sc_gather_body)
    return k(tbl, idx)


_NBUF = 4
_NBLK = N_VOXEL // _TC_B


def _tc_expand_body(g_ref, vl_ref, out_hbm, buf_ref, sems):
    i = pl.program_id(0)
    slot = i % _NBUF
    seq = g_ref[:, :D]
    val = g_ref[:, D_VAL:D_VAL + D]
    vl = vl_ref[...]
    col = lax.broadcasted_iota(jnp.int32, (seq.shape[0], D), 1)
    scaled = ((col >= N_SPECIES) & (col < 2 * N_SPECIES)) | (col >= 5 * N_SPECIES)
    val = val * jnp.where(scaled, jnp.float32(1.0 / 254.0), jnp.float32(1.0))
    val = jnp.where(vl != -1, val, jnp.float32(0.0))
    seqi = seq.astype(jnp.int32)

    @pl.when(i >= _NBUF)
    def _():
        pltpu.make_async_copy(
            buf_ref.at[slot],
            out_hbm.at[pl.ds((i - _NBUF) * _TC_B, _TC_B), :],
            sems.at[slot],
        ).wait()

    for a in range(N_AA):
        buf_ref[slot, :, a * D:(a + 1) * D] = jnp.where(
            seqi == a, val, jnp.float32(0.0))

    pltpu.make_async_copy(
        buf_ref.at[slot],
        out_hbm.at[pl.ds(i * _TC_B, _TC_B), :],
        sems.at[slot],
    ).start()

    @pl.when(i == _NBLK - 1)
    def _():
        for k in range(_NBUF):
            j = _NBLK - _NBUF + k
            pltpu.make_async_copy(
                buf_ref.at[j % _NBUF],
                out_hbm.at[pl.ds(j * _TC_B, _TC_B), :],
                sems.at[j % _NBUF],
            ).wait()


def _tc_expand(g, vl2d):
    return pl.pallas_call(
        _tc_expand_body,
        grid=(_NBLK,),
        in_specs=[
            pl.BlockSpec((_TC_B, D_TBL), lambda i: (i, 0)),
            pl.BlockSpec((_TC_B, 1), lambda i: (i, 0)),
        ],
        out_specs=pl.BlockSpec(memory_space=pl.ANY),
        out_shape=jax.ShapeDtypeStruct((N_VOXEL, D_OUT_PAD), jnp.float32),
        scratch_shapes=[
            pltpu.VMEM((_NBUF, _TC_B, D_OUT_PAD), jnp.float32),
            pltpu.SemaphoreType.DMA((_NBUF,)),
        ],
    )(g, vl2d)


def kernel(seqArr, gapArr, stopArr, globalArr, voxel_local):
    seqf = seqArr.astype(jnp.float32)
    g = globalArr[0]
    ones = jnp.ones((N_POS, N_SPECIES), jnp.float32)
    z24 = jnp.zeros((N_POS, D_VAL - D), jnp.float32)
    tbl = jnp.concatenate(
        [
            seqf, seqf, seqf, seqf, seqf, seqf,
            z24,
            ones,
            gapArr,
            stopArr,
            jnp.broadcast_to(g[0][None, :], (N_POS, N_SPECIES)),
            jnp.broadcast_to(g[1][None, :], (N_POS, N_SPECIES)),
            jnp.broadcast_to(g[2][None, :], (N_POS, N_SPECIES)),
            z24,
        ],
        axis=1,
    )
    safe_idx = jnp.clip(voxel_local, 0, N_POS - 1)
    gathered = _sc_gather(tbl, safe_idx)
    vl2d = voxel_local.reshape(N_VOXEL, 1)
    padded = _tc_expand(gathered, vl2d)
    return padded[:, :D_OUT]


# speedup vs baseline: 1.7103x; 1.0252x over previous
import functools

import jax
import jax.numpy as jnp
from jax import lax
from jax.experimental import pallas as pl
from jax.experimental.pallas import tpu as pltpu
from jax.experimental.pallas import tpu_sc as plsc

N_POS = 2048
N_SPECIES = 60
N_AA = 23
N_VOXEL = 4096
D = 6 * N_SPECIES
D_VAL = 384
D_TBL = 768
D_OUT = N_AA * D
D_OUT_PAD = 8320

_NC = 2
_NS = 16
_NW = _NC * _NS
_BPW = N_VOXEL // _NW

_TC_B = 256


def _sc_gather_body(tbl_hbm, idx_hbm, out_hbm, idx_v, rows_v, sem):
    wid = lax.axis_index("s") * _NC + lax.axis_index("c")
    base = wid * _BPW
    pltpu.sync_copy(idx_hbm.at[pl.ds(base, _BPW)], idx_v)
    pltpu.async_copy(tbl_hbm.at[idx_v], rows_v, sem).wait()
    pltpu.sync_copy(rows_v, out_hbm.at[pl.ds(base, _BPW)])


def _sc_gather(tbl, idx):
    mesh = plsc.VectorSubcoreMesh(core_axis_name="c", subcore_axis_name="s")
    k = functools.partial(
        pl.kernel,
        mesh=mesh,
        out_type=jax.ShapeDtypeStruct((N_VOXEL, D_TBL), jnp.float32),
        scratch_types=[
            pltpu.VMEM((_BPW,), jnp.int32),
            pltpu.VMEM((_BPW, D_TBL), jnp.float32),
            pltpu.SemaphoreType.DMA,
        ],
    )(_sc_gather_body)
    return k(tbl, idx)


_NBUF = 3
_NBLK = N_VOXEL // _TC_B


def _tc_expand_body(g_ref, vl_ref, out_hbm, buf_ref, sems):
    i = pl.program_id(0)
    slot = i % _NBUF
    seq = g_ref[:, :D]
    val = g_ref[:, D_VAL:D_VAL + D]
    vl = vl_ref[...]
    col = lax.broadcasted_iota(jnp.int32, (seq.shape[0], D), 1)
    scaled = ((col >= N_SPECIES) & (col < 2 * N_SPECIES)) | (col >= 5 * N_SPECIES)
    val = val * jnp.where(scaled, jnp.float32(1.0 / 254.0), jnp.float32(1.0))
    val = jnp.where(vl != -1, val, jnp.float32(0.0))
    seqi = seq.astype(jnp.int32)

    @pl.when(i >= _NBUF)
    def _():
        pltpu.make_async_copy(
            buf_ref.at[slot],
            out_hbm.at[pl.ds((i - _NBUF) * _TC_B, _TC_B), :],
            sems.at[slot],
        ).wait()

    for a in range(N_AA):
        buf_ref[slot, :, a * D:(a + 1) * D] = jnp.where(
            seqi == a, val, jnp.float32(0.0))

    pltpu.make_async_copy(
        buf_ref.at[slot],
        out_hbm.at[pl.ds(i * _TC_B, _TC_B), :],
        sems.at[slot],
    ).start()

    @pl.when(i == _NBLK - 1)
    def _():
        for k in range(_NBUF):
            j = _NBLK - _NBUF + k
            pltpu.make_async_copy(
                buf_ref.at[j % _NBUF],
                out_hbm.at[pl.ds(j * _TC_B, _TC_B), :],
                sems.at[j % _NBUF],
            ).wait()


def _tc_expand(g, vl2d):
    return pl.pallas_call(
        _tc_expand_body,
        grid=(_NBLK,),
        in_specs=[
            pl.BlockSpec((_TC_B, D_TBL), lambda i: (i, 0)),
            pl.BlockSpec((_TC_B, 1), lambda i: (i, 0)),
        ],
        out_specs=pl.BlockSpec(memory_space=pl.ANY),
        out_shape=jax.ShapeDtypeStruct((N_VOXEL, D_OUT_PAD), jnp.float32),
        scratch_shapes=[
            pltpu.VMEM((_NBUF, _TC_B, D_OUT_PAD), jnp.float32),
            pltpu.SemaphoreType.DMA((_NBUF,)),
        ],
    )(g, vl2d)


def kernel(seqArr, gapArr, stopArr, globalArr, voxel_local):
    seqf = seqArr.astype(jnp.float32)
    g = globalArr[0]
    ones = jnp.ones((N_POS, N_SPECIES), jnp.float32)
    z24 = jnp.zeros((N_POS, D_VAL - D), jnp.float32)
    tbl = jnp.concatenate(
        [
            seqf, seqf, seqf, seqf, seqf, seqf,
            z24,
            ones,
            gapArr,
            stopArr,
            jnp.broadcast_to(g[0][None, :], (N_POS, N_SPECIES)),
            jnp.broadcast_to(g[1][None, :], (N_POS, N_SPECIES)),
            jnp.broadcast_to(g[2][None, :], (N_POS, N_SPECIES)),
            z24,
        ],
        axis=1,
    )
    safe_idx = jnp.clip(voxel_local, 0, N_POS - 1)
    gathered = _sc_gather(tbl, safe_idx)
    vl2d = voxel_local.reshape(N_VOXEL, 1)
    padded = _tc_expand(gathered, vl2d)
    return padded[:, :D_OUT]
